# Initial kernel scaffold; baseline (speedup 1.0000x reference)
import jax
import jax.numpy as jnp
from jax.experimental import pallas as pl


def kernel(x, edge_index, op_mask, f_init_W1, f_init_b1, f_init_W2, f_init_b2, sage0_Wself, sage0_Wneigh, sage0_b, fs0_W1, fs0_b1, fs0_W2, fs0_b2, sage1_Wself, sage1_Wneigh, sage1_b, fs1_W1, fs1_b1, fs1_W2, fs1_b2, final_W1, final_b1, final_W2, final_b2):
    raise NotImplementedError("write your pallas kernel here")


# speedup vs baseline: 4.8771x; 4.8771x over previous
import functools

import jax
import jax.numpy as jnp
import numpy as np
from jax import lax
from jax.experimental import pallas as pl
from jax.experimental.pallas import tpu as pltpu
from jax.experimental.pallas import tpu_sc as plsc

_N = 10000
_E = 320000
_D = 128
_NC = 2
_NS = 16
_NW = _NC * _NS
_EPW = _E // _NW
_K = 80
_NACC = 10240
_RPT = _NACC // _NS
_R = 1024
_F32 = jnp.float32


def _make_sc_scatter(with_counts):
    mesh = plsc.VectorSubcoreMesh(core_axis_name="c", subcore_axis_name="s")
    out_type = [jax.ShapeDtypeStruct((_NC, _NACC, _D), _F32)]
    scratch = [
        pltpu.VMEM((_K,), jnp.int32),
        pltpu.VMEM((_K,), jnp.int32),
        pltpu.VMEM((_K, _D), _F32),
        pltpu.VMEM_SHARED((_NACC, _D), _F32),
        pltpu.SemaphoreType.DMA,
    ]
    if with_counts:
        out_type.append(jax.ShapeDtypeStruct((_NC, _NS, _NACC), _F32))
        scratch.insert(4, pltpu.VMEM((_NACC,), _F32))

    @functools.partial(
        pl.kernel, mesh=mesh, out_type=out_type, scratch_types=scratch,
        compiler_params=pltpu.CompilerParams(needs_layout_passes=False))
    def sc_fn(table, srcl, dstl, zrows, zcnt, *rest):
        if with_counts:
            out, out_cnt, src_v, dst_v, rows_v, acc, cnt_v, sem = rest
        else:
            out, src_v, dst_v, rows_v, acc, sem = rest
        c = lax.axis_index("c")
        s = lax.axis_index("s")
        wid = s * _NC + c
        pltpu.sync_copy(zrows, acc.at[pl.ds(s * _RPT, _RPT), :])
        if with_counts:
            pltpu.sync_copy(zcnt, cnt_v)
        plsc.subcore_barrier()
        base = wid * _EPW
        ones16 = jnp.full((16,), 1.0, _F32)

        def body(i, carry):
            off = base + i * _K
            pltpu.sync_copy(srcl.at[pl.ds(off, _K)], src_v)
            pltpu.sync_copy(dstl.at[pl.ds(off, _K)], dst_v)
            pltpu.async_copy(table.at[src_v], rows_v, sem).wait()
            pltpu.sync_copy(rows_v, acc.at[dst_v], add=True)
            if with_counts:
                for j in range(_K // 16):
                    idx = dst_v[pl.ds(j * 16, 16)]
                    plsc.addupdate_scatter(cnt_v, [idx], ones16)
            return carry

        lax.fori_loop(0, _EPW // _K, body, 0)
        plsc.subcore_barrier()
        pltpu.sync_copy(acc.at[pl.ds(s * _RPT, _RPT), :],
                        out.at[c, pl.ds(s * _RPT, _RPT), :])
        if with_counts:
            pltpu.sync_copy(cnt_v, out_cnt.at[c, s, :])

    return sc_fn


_sc_scatter_cnt = _make_sc_scatter(True)
_sc_scatter = _make_sc_scatter(False)


def _k_init(x_ref, w1, b1, w2, b2, wn, h_ref, t0_ref):
    hh = jnp.maximum(x_ref[...] @ w1[...] + b1[...], 0.0) @ w2[...] + b2[...]
    h_ref[...] = hh
    t0_ref[...] = hh @ wn[...]


def _k_mid0(h_ref, p_ref, c_ref, wself, b, w1, b1, w2, b2, wn1,
            h1_ref, t1_ref, inv_ref):
    pp = p_ref[0] + p_ref[1]
    cnt = jnp.sum(c_ref[...].reshape(_NW, _R), axis=0)[:, None]
    inv = 1.0 / jnp.maximum(cnt, 1.0)
    s = h_ref[...] @ wself[...] + pp * inv + b[...]
    h1 = jnp.maximum(s @ w1[...] + b1[...], 0.0) @ w2[...] + b2[...]
    h1_ref[...] = h1
    t1_ref[...] = h1 @ wn1[...]
    inv_ref[...] = jnp.broadcast_to(inv, (_R, _D))


def _k_mid1(h_ref, q_ref, inv_ref, wself, b, w1, b1, w2, b2, h2_ref, cs_ref):
    agg = (q_ref[0] + q_ref[1]) * inv_ref[...]
    s = h_ref[...] @ wself[...] + agg + b[...]
    h2 = jnp.maximum(s @ w1[...] + b1[...], 0.0) @ w2[...] + b2[...]
    h2_ref[...] = h2

    @pl.when(pl.program_id(0) == 0)
    def _():
        cs_ref[...] = jnp.zeros_like(cs_ref)

    rid = pl.program_id(0) * _R + lax.broadcasted_iota(jnp.int32, (_R, 1), 0)
    cs_ref[...] += jnp.sum(jnp.where(rid < _N, h2, 0.0), axis=0, keepdims=True)


def _k_final(h2_ref, g_ref, w1, b1, w2, b2, lm_ref, z_ref):
    w = w1[...]
    gv = g_ref[...] @ w[_D:, :] + b1[...]
    t = jnp.maximum(h2_ref[...] @ w[:_D, :] + gv, 0.0)
    z_ref[...] = t @ w2[...] + b2[...] + lm_ref[...]


def _k_softmax(z_ref, o_ref):
    z = z_ref[...]
    m = jnp.max(z)
    e = jnp.exp(z - m)
    o_ref[...] = e / jnp.sum(e)


def _full(shape):
    return pl.BlockSpec(shape, lambda i: tuple(0 for _ in shape))


def _rows(width):
    return pl.BlockSpec((_R, width), lambda i: (i, 0))


_GRID = (pl.cdiv(_N, _R),)


def _tc_init(x, w1, b1, w2, b2, wn):
    return pl.pallas_call(
        _k_init,
        grid=_GRID,
        in_specs=[_rows(_D), _full((_D, _D)), _full((1, _D)),
                  _full((_D, _D)), _full((1, _D)), _full((_D, _D))],
        out_specs=[_rows(_D), _rows(_D)],
        out_shape=[jax.ShapeDtypeStruct((_N, _D), _F32)] * 2,
    )(x, w1, b1, w2, b2, wn)


def _tc_mid0(h, p, cnts, wself, b, w1, b1, w2, b2, wn1):
    return pl.pallas_call(
        _k_mid0,
        grid=_GRID,
        in_specs=[_rows(_D),
                  pl.BlockSpec((_NC, _R, _D), lambda i: (0, i, 0)),
                  pl.BlockSpec((_NC, _NS, _R), lambda i: (0, 0, i)),
                  _full((_D, _D)), _full((1, _D)), _full((_D, _D)),
                  _full((1, _D)), _full((_D, _D)), _full((1, _D)),
                  _full((_D, _D))],
        out_specs=[_rows(_D), _rows(_D), _rows(_D)],
        out_shape=[jax.ShapeDtypeStruct((_N, _D), _F32)] * 3,
    )(h, p, cnts, wself, b, w1, b1, w2, b2, wn1)


def _tc_mid1(h1, q, inv, wself, b, w1, b1, w2, b2):
    return pl.pallas_call(
        _k_mid1,
        grid=_GRID,
        in_specs=[_rows(_D),
                  pl.BlockSpec((_NC, _R, _D), lambda i: (0, i, 0)),
                  _rows(_D),
                  _full((_D, _D)), _full((1, _D)), _full((_D, _D)),
                  _full((1, _D)), _full((_D, _D)), _full((1, _D))],
        out_specs=[_rows(_D), pl.BlockSpec((1, _D), lambda i: (0, 0))],
        out_shape=[jax.ShapeDtypeStruct((_N, _D), _F32),
                   jax.ShapeDtypeStruct((1, _D), _F32)],
    )(h1, q, inv, wself, b, w1, b1, w2, b2)


def _tc_final(h2, g, w1, b1, w2, b2, lm):
    return pl.pallas_call(
        _k_final,
        grid=_GRID,
        in_specs=[_rows(_D), _full((1, _D)), _full((2 * _D, 2 * _D)),
                  _full((1, 2 * _D)), _full((2 * _D, 1)), _full((1, 1)),
                  _rows(1)],
        out_specs=_rows(1),
        out_shape=jax.ShapeDtypeStruct((_N, 1), _F32),
    )(h2, g, w1, b1, w2, b2, lm)


_NPAD = 79 * 128


def _tc_softmax(zp):
    return pl.pallas_call(
        _k_softmax,
        in_specs=[pl.BlockSpec((79, 128), lambda: (0, 0))],
        out_specs=pl.BlockSpec((79, 128), lambda: (0, 0)),
        out_shape=jax.ShapeDtypeStruct((79, 128), _F32),
    )(zp)


def kernel(x, edge_index, op_mask, f_init_W1, f_init_b1, f_init_W2, f_init_b2,
           sage0_Wself, sage0_Wneigh, sage0_b, fs0_W1, fs0_b1, fs0_W2, fs0_b2,
           sage1_Wself, sage1_Wneigh, sage1_b, fs1_W1, fs1_b1, fs1_W2, fs1_b2,
           final_W1, final_b1, final_W2, final_b2):
    src = edge_index[0].astype(jnp.int32)
    dst = edge_index[1].astype(jnp.int32)
    row = lambda v: v.reshape(1, -1)
    zrows = jnp.zeros((_RPT, _D), _F32)
    zcnt = jnp.zeros((_NACC,), _F32)

    h, t0 = _tc_init(x, f_init_W1, row(f_init_b1), f_init_W2, row(f_init_b2),
                     sage0_Wneigh)

    p0, cnts = _sc_scatter_cnt(t0, src, dst, zrows, zcnt)

    h1, t1, inv = _tc_mid0(h, p0, cnts, sage0_Wself, row(sage0_b), fs0_W1,
                           row(fs0_b1), fs0_W2, row(fs0_b2), sage1_Wneigh)

    (q,) = _sc_scatter(t1, src, dst, zrows, zcnt)

    h2, cs = _tc_mid1(h1, q, inv, sage1_Wself, row(sage1_b), fs1_W1,
                      row(fs1_b1), fs1_W2, row(fs1_b2))

    g = cs / np.float32(_N)
    lm = jnp.log(op_mask.astype(_F32)).reshape(-1, 1)
    z = _tc_final(h2, g, final_W1, row(final_b1), final_W2,
                  final_b2.reshape(1, 1), lm)

    zp = jnp.pad(z.reshape(-1), (0, _NPAD - _N),
                 constant_values=-1e30).reshape(79, 128)
    p = _tc_softmax(zp)
    return p.reshape(-1)[:_N].reshape(_N, 1)


# speedup vs baseline: 8.9756x; 1.8404x over previous
import functools

import jax
import jax.numpy as jnp
import numpy as np
from jax import lax
from jax.experimental import pallas as pl
from jax.experimental.pallas import tpu as pltpu
from jax.experimental.pallas import tpu_sc as plsc

_N = 10000
_E = 320000
_D = 128
_NC = 2
_NS = 16
_NW = _NC * _NS
_EPW = _E // _NW
_K = 80
_NACC = 10240
_RPT = _NACC // _NS
_R = 1024
_F32 = jnp.float32


def _make_sc_scatter(with_counts):
    mesh = plsc.VectorSubcoreMesh(core_axis_name="c", subcore_axis_name="s")
    out_type = [jax.ShapeDtypeStruct((_NC, _NACC, _D), _F32)]
    scratch = [
        pltpu.VMEM((_K,), jnp.int32),
        pltpu.VMEM((_K,), jnp.int32),
        pltpu.VMEM((_K,), jnp.int32),
        pltpu.VMEM((_K,), jnp.int32),
        pltpu.VMEM((_K, _D), _F32),
        pltpu.VMEM((_K, _D), _F32),
        pltpu.VMEM_SHARED((_NACC, _D), _F32),
        pltpu.SemaphoreType.DMA,
        pltpu.SemaphoreType.DMA,
        pltpu.SemaphoreType.DMA,
        pltpu.SemaphoreType.DMA,
    ]
    if with_counts:
        out_type.append(jax.ShapeDtypeStruct((_NC, _NS, _NACC), _F32))
        scratch.append(pltpu.VMEM((_NACC,), _F32))
    nch = _EPW // _K

    @functools.partial(
        pl.kernel, mesh=mesh, out_type=out_type, scratch_types=scratch,
        compiler_params=pltpu.CompilerParams(needs_layout_passes=False))
    def sc_fn(table, srcl, dstl, zrows, zcnt, *rest):
        if with_counts:
            (out, out_cnt, src0, src1, dst0, dst1, rows0, rows1, acc,
             gsem0, gsem1, isem0, isem1, cnt_v) = rest
        else:
            (out, src0, src1, dst0, dst1, rows0, rows1, acc,
             gsem0, gsem1, isem0, isem1) = rest
        src_v = (src0, src1)
        dst_v = (dst0, dst1)
        rows_v = (rows0, rows1)
        gsem = (gsem0, gsem1)
        isem = (isem0, isem1)
        c = lax.axis_index("c")
        s = lax.axis_index("s")
        wid = s * _NC + c
        pltpu.sync_copy(zrows, acc.at[pl.ds(s * _RPT, _RPT), :])
        if with_counts:
            pltpu.sync_copy(zcnt, cnt_v)
        plsc.subcore_barrier()
        base = wid * _EPW
        ones16 = jnp.full((16,), 1.0, _F32)

        def load_idx(i, b):
            off = base + i * _K
            pltpu.async_copy(srcl.at[pl.ds(off, _K)], src_v[b], isem[b])
            pltpu.async_copy(dstl.at[pl.ds(off, _K)], dst_v[b], isem[b])

        def hist(b):
            if with_counts:
                for j in range(_K // 16):
                    idx = dst_v[b][pl.ds(j * 16, 16)]
                    plsc.addupdate_scatter(cnt_v, [idx], ones16)

        off0 = base
        pltpu.sync_copy(srcl.at[pl.ds(off0, _K)], src0)
        pltpu.sync_copy(dstl.at[pl.ds(off0, _K)], dst0)
        pltpu.async_copy(table.at[src0], rows0, gsem0)
        load_idx(1, 1)

        def body(t, carry):
            for b in (0, 1):
                i = 2 * t + b
                o = 1 - b
                pltpu.make_async_copy(srcl.at[pl.ds(0, _K)], src_v[o],
                                      isem[o]).wait()
                pltpu.make_async_copy(dstl.at[pl.ds(0, _K)], dst_v[o],
                                      isem[o]).wait()
                pltpu.async_copy(table.at[src_v[o]], rows_v[o], gsem[o])
                hist(b)
                pltpu.make_async_copy(table.at[src_v[b]], rows_v[b],
                                      gsem[b]).wait()
                pltpu.sync_copy(rows_v[b], acc.at[dst_v[b]], add=True)

                @pl.when(i + 2 < nch)
                def _():
                    load_idx(i + 2, b)
            return carry

        lax.fori_loop(0, (nch - 1) // 2, body, 0)
        hist(0)
        pltpu.make_async_copy(table.at[src0], rows0, gsem0).wait()
        pltpu.sync_copy(rows0, acc.at[dst0], add=True)

        plsc.subcore_barrier()
        pltpu.sync_copy(acc.at[pl.ds(s * _RPT, _RPT), :],
                        out.at[c, pl.ds(s * _RPT, _RPT), :])
        if with_counts:
            pltpu.sync_copy(cnt_v, out_cnt.at[c, s, :])

    return sc_fn


_sc_scatter_cnt = _make_sc_scatter(True)
_sc_scatter = _make_sc_scatter(False)


def _k_init(x_ref, w1, b1, w2, b2, wn, h_ref, t0_ref):
    hh = jnp.maximum(x_ref[...] @ w1[...] + b1[...], 0.0) @ w2[...] + b2[...]
    h_ref[...] = hh
    t0_ref[...] = hh @ wn[...]


def _k_mid0(h_ref, p_ref, c_ref, wself, b, w1, b1, w2, b2, wn1,
            h1_ref, t1_ref, inv_ref):
    pp = p_ref[0] + p_ref[1]
    cnt = jnp.sum(c_ref[...].reshape(_NW, _R), axis=0)[:, None]
    inv = 1.0 / jnp.maximum(cnt, 1.0)
    s = h_ref[...] @ wself[...] + pp * inv + b[...]
    h1 = jnp.maximum(s @ w1[...] + b1[...], 0.0) @ w2[...] + b2[...]
    h1_ref[...] = h1
    t1_ref[...] = h1 @ wn1[...]
    inv_ref[...] = jnp.broadcast_to(inv, (_R, _D))


def _k_mid1(h_ref, q_ref, inv_ref, wself, b, w1, b1, w2, b2, h2_ref, cs_ref):
    agg = (q_ref[0] + q_ref[1]) * inv_ref[...]
    s = h_ref[...] @ wself[...] + agg + b[...]
    h2 = jnp.maximum(s @ w1[...] + b1[...], 0.0) @ w2[...] + b2[...]
    h2_ref[...] = h2

    @pl.when(pl.program_id(0) == 0)
    def _():
        cs_ref[...] = jnp.zeros_like(cs_ref)

    rid = pl.program_id(0) * _R + lax.broadcasted_iota(jnp.int32, (_R, 1), 0)
    cs_ref[...] += jnp.sum(jnp.where(rid < _N, h2, 0.0), axis=0, keepdims=True)


def _k_final(h2_ref, g_ref, w1, b1, w2, b2, lm_ref, z_ref):
    w = w1[...]
    gv = g_ref[...] @ w[_D:, :] + b1[...]
    t = jnp.maximum(h2_ref[...] @ w[:_D, :] + gv, 0.0)
    z_ref[...] = t @ w2[...] + b2[...] + lm_ref[...]


def _k_softmax(z_ref, o_ref):
    z = z_ref[...]
    m = jnp.max(z)
    e = jnp.exp(z - m)
    o_ref[...] = e / jnp.sum(e)


def _full(shape):
    return pl.BlockSpec(shape, lambda i: tuple(0 for _ in shape))


def _rows(width):
    return pl.BlockSpec((_R, width), lambda i: (i, 0))


_GRID = (pl.cdiv(_N, _R),)


def _tc_init(x, w1, b1, w2, b2, wn):
    return pl.pallas_call(
        _k_init,
        grid=_GRID,
        in_specs=[_rows(_D), _full((_D, _D)), _full((1, _D)),
                  _full((_D, _D)), _full((1, _D)), _full((_D, _D))],
        out_specs=[_rows(_D), _rows(_D)],
        out_shape=[jax.ShapeDtypeStruct((_N, _D), _F32)] * 2,
    )(x, w1, b1, w2, b2, wn)


def _tc_mid0(h, p, cnts, wself, b, w1, b1, w2, b2, wn1):
    return pl.pallas_call(
        _k_mid0,
        grid=_GRID,
        in_specs=[_rows(_D),
                  pl.BlockSpec((_NC, _R, _D), lambda i: (0, i, 0)),
                  pl.BlockSpec((_NC, _NS, _R), lambda i: (0, 0, i)),
                  _full((_D, _D)), _full((1, _D)), _full((_D, _D)),
                  _full((1, _D)), _full((_D, _D)), _full((1, _D)),
                  _full((_D, _D))],
        out_specs=[_rows(_D), _rows(_D), _rows(_D)],
        out_shape=[jax.ShapeDtypeStruct((_N, _D), _F32)] * 3,
    )(h, p, cnts, wself, b, w1, b1, w2, b2, wn1)


def _tc_mid1(h1, q, inv, wself, b, w1, b1, w2, b2):
    return pl.pallas_call(
        _k_mid1,
        grid=_GRID,
        in_specs=[_rows(_D),
                  pl.BlockSpec((_NC, _R, _D), lambda i: (0, i, 0)),
                  _rows(_D),
                  _full((_D, _D)), _full((1, _D)), _full((_D, _D)),
                  _full((1, _D)), _full((_D, _D)), _full((1, _D))],
        out_specs=[_rows(_D), pl.BlockSpec((1, _D), lambda i: (0, 0))],
        out_shape=[jax.ShapeDtypeStruct((_N, _D), _F32),
                   jax.ShapeDtypeStruct((1, _D), _F32)],
    )(h1, q, inv, wself, b, w1, b1, w2, b2)


def _tc_final(h2, g, w1, b1, w2, b2, lm):
    return pl.pallas_call(
        _k_final,
        grid=_GRID,
        in_specs=[_rows(_D), _full((1, _D)), _full((2 * _D, 2 * _D)),
                  _full((1, 2 * _D)), _full((2 * _D, 1)), _full((1, 1)),
                  _rows(1)],
        out_specs=_rows(1),
        out_shape=jax.ShapeDtypeStruct((_N, 1), _F32),
    )(h2, g, w1, b1, w2, b2, lm)


_NPAD = 79 * 128


def _tc_softmax(zp):
    return pl.pallas_call(
        _k_softmax,
        in_specs=[pl.BlockSpec((79, 128), lambda: (0, 0))],
        out_specs=pl.BlockSpec((79, 128), lambda: (0, 0)),
        out_shape=jax.ShapeDtypeStruct((79, 128), _F32),
    )(zp)


def kernel(x, edge_index, op_mask, f_init_W1, f_init_b1, f_init_W2, f_init_b2,
           sage0_Wself, sage0_Wneigh, sage0_b, fs0_W1, fs0_b1, fs0_W2, fs0_b2,
           sage1_Wself, sage1_Wneigh, sage1_b, fs1_W1, fs1_b1, fs1_W2, fs1_b2,
           final_W1, final_b1, final_W2, final_b2):
    src = edge_index[0].astype(jnp.int32)
    dst = edge_index[1].astype(jnp.int32)
    row = lambda v: v.reshape(1, -1)
    zrows = jnp.zeros((_RPT, _D), _F32)
    zcnt = jnp.zeros((_NACC,), _F32)

    h, t0 = _tc_init(x, f_init_W1, row(f_init_b1), f_init_W2, row(f_init_b2),
                     sage0_Wneigh)

    p0, cnts = _sc_scatter_cnt(t0, src, dst, zrows, zcnt)

    h1, t1, inv = _tc_mid0(h, p0, cnts, sage0_Wself, row(sage0_b), fs0_W1,
                           row(fs0_b1), fs0_W2, row(fs0_b2), sage1_Wneigh)

    (q,) = _sc_scatter(t1, src, dst, zrows, zcnt)

    h2, cs = _tc_mid1(h1, q, inv, sage1_Wself, row(sage1_b), fs1_W1,
                      row(fs1_b1), fs1_W2, row(fs1_b2))

    g = cs / np.float32(_N)
    lm = jnp.log(op_mask.astype(_F32)).reshape(-1, 1)
    z = _tc_final(h2, g, final_W1, row(final_b1), final_W2,
                  final_b2.reshape(1, 1), lm)

    zp = jnp.pad(z.reshape(-1), (0, _NPAD - _N),
                 constant_values=-1e30).reshape(79, 128)
    p = _tc_softmax(zp)
    return p.reshape(-1)[:_N].reshape(_N, 1)


# speedup vs baseline: 10.1264x; 1.1282x over previous
import functools

import jax
import jax.numpy as jnp
import numpy as np
from jax import lax
from jax.experimental import pallas as pl
from jax.experimental.pallas import tpu as pltpu
from jax.experimental.pallas import tpu_sc as plsc

_N = 10000
_E = 320000
_D = 128
_NC = 2
_NS = 16
_NW = _NC * _NS
_EPW = _E // _NW
_K = 80
_NACC = 10240
_RPT = _NACC // _NS
_R = 1024
_F32 = jnp.float32


def _make_sc_scatter(with_counts):
    mesh = plsc.VectorSubcoreMesh(core_axis_name="c", subcore_axis_name="s")
    out_type = [jax.ShapeDtypeStruct((_NC, _NACC, _D), _F32)]
    scratch = (
        [pltpu.VMEM((_K,), jnp.int32)] * 2
        + [pltpu.VMEM((_K,), jnp.int32)] * 4
        + [pltpu.VMEM((_K, _D), _F32)] * 2
        + [pltpu.VMEM_SHARED((_NACC, _D), _F32)]
        + [pltpu.SemaphoreType.DMA] * 10
    )
    if with_counts:
        out_type.append(jax.ShapeDtypeStruct((_NC, _NS, _NACC), _F32))
        scratch.append(pltpu.VMEM((_NACC,), _F32))
    nch = _EPW // _K
    nmain = (nch - 1) // 4 * 4

    @functools.partial(
        pl.kernel, mesh=mesh, out_type=out_type, scratch_types=scratch,
        compiler_params=pltpu.CompilerParams(needs_layout_passes=False))
    def sc_fn(table, srcl, dstl, zrows, zcnt, *rest):
        if with_counts:
            out, out_cnt = rest[0], rest[1]
            rest = rest[2:]
            cnt_v = rest[-1]
        else:
            out = rest[0]
            rest = rest[1:]
        src_v = rest[0:2]
        dst_v = rest[2:6]
        rows_v = rest[6:8]
        acc = rest[8]
        gsem = rest[9:11]
        ssem = rest[11:13]
        isrc = rest[13:15]
        idst = rest[15:19]
        c = lax.axis_index("c")
        s = lax.axis_index("s")
        wid = s * _NC + c
        pltpu.sync_copy(zrows, acc.at[pl.ds(s * _RPT, _RPT), :])
        if with_counts:
            pltpu.sync_copy(zcnt, cnt_v)
        plsc.subcore_barrier()
        base = wid * _EPW
        ones16 = jnp.full((16,), 1.0, _F32)

        def load_idx(i, b, u):
            off = base + i * _K
            pltpu.async_copy(srcl.at[pl.ds(off, _K)], src_v[b], isrc[b])
            pltpu.async_copy(dstl.at[pl.ds(off, _K)], dst_v[u], idst[u])

        def hist(u):
            if with_counts:
                for j in range(_K // 16):
                    idx = dst_v[u][pl.ds(j * 16, 16)]
                    plsc.addupdate_scatter(cnt_v, [idx], ones16)

        def wait_idx(b, u):
            pltpu.make_async_copy(srcl.at[pl.ds(0, _K)], src_v[b],
                                  isrc[b]).wait()
            pltpu.make_async_copy(dstl.at[pl.ds(0, _K)], dst_v[u],
                                  idst[u]).wait()

        def wait_gather(b):
            pltpu.make_async_copy(table.at[src_v[b]], rows_v[b],
                                  gsem[b]).wait()

        def wait_scatter(b):
            pltpu.make_async_copy(rows_v[b], acc.at[dst_v[0]],
                                  ssem[b]).wait()

        pltpu.sync_copy(srcl.at[pl.ds(base, _K)], src_v[0])
        pltpu.sync_copy(dstl.at[pl.ds(base, _K)], dst_v[0])
        pltpu.async_copy(table.at[src_v[0]], rows_v[0], gsem[0])
        load_idx(1, 1, 1)

        def chunk(i, u, first=False):
            b = u % 2
            o = 1 - b
            wait_idx(o, (u + 1) % 4)
            if not first:
                wait_scatter(o)
            pltpu.async_copy(table.at[src_v[o]], rows_v[o], gsem[o])
            hist(u)
            wait_gather(b)
            pltpu.async_copy(rows_v[b], acc.at[dst_v[u]], ssem[b], add=True)

            @pl.when(i + 2 < nch)
            def _():
                load_idx(i + 2, b, (u + 2) % 4)

        def body(t, carry):
            for u in (0, 1, 2, 3):
                chunk(4 * t + u, u, first=False)
            return carry

        for u in (0, 1, 2, 3):
            chunk(u, u, first=(u == 0))
        lax.fori_loop(1, nmain // 4, body, 0)
        wait_scatter(1)
        hist(0)
        wait_gather(0)
        pltpu.async_copy(rows_v[0], acc.at[dst_v[0]], ssem[0], add=True)
        wait_scatter(0)

        plsc.subcore_barrier()
        pltpu.sync_copy(acc.at[pl.ds(s * _RPT, _RPT), :],
                        out.at[c, pl.ds(s * _RPT, _RPT), :])
        if with_counts:
            pltpu.sync_copy(cnt_v, out_cnt.at[c, s, :])

    return sc_fn


_sc_scatter_cnt = _make_sc_scatter(True)
_sc_scatter = _make_sc_scatter(False)


def _k_init(x_ref, w1, b1, w2, b2, wn, h_ref, t0_ref):
    hh = jnp.maximum(x_ref[...] @ w1[...] + b1[...], 0.0) @ w2[...] + b2[...]
    h_ref[...] = hh
    t0_ref[...] = hh @ wn[...]


def _k_mid0(h_ref, p_ref, c_ref, wself, b, w1, b1, w2, b2, wn1,
            h1_ref, t1_ref, inv_ref):
    pp = p_ref[0] + p_ref[1]
    cnt = jnp.sum(c_ref[...].reshape(_NW, _R), axis=0)[:, None]
    inv = 1.0 / jnp.maximum(cnt, 1.0)
    s = h_ref[...] @ wself[...] + pp * inv + b[...]
    h1 = jnp.maximum(s @ w1[...] + b1[...], 0.0) @ w2[...] + b2[...]
    h1_ref[...] = h1
    t1_ref[...] = h1 @ wn1[...]
    inv_ref[...] = jnp.broadcast_to(inv, (_R, _D))


def _k_mid1(h_ref, q_ref, inv_ref, wself, b, w1, b1, w2, b2, h2_ref, cs_ref):
    agg = (q_ref[0] + q_ref[1]) * inv_ref[...]
    s = h_ref[...] @ wself[...] + agg + b[...]
    h2 = jnp.maximum(s @ w1[...] + b1[...], 0.0) @ w2[...] + b2[...]
    h2_ref[...] = h2

    @pl.when(pl.program_id(0) == 0)
    def _():
        cs_ref[...] = jnp.zeros_like(cs_ref)

    rid = pl.program_id(0) * _R + lax.broadcasted_iota(jnp.int32, (_R, 1), 0)
    cs_ref[...] += jnp.sum(jnp.where(rid < _N, h2, 0.0), axis=0, keepdims=True)


def _k_final(h2_ref, g_ref, w1, b1, w2, b2, lm_ref, z_ref):
    w = w1[...]
    gv = g_ref[...] @ w[_D:, :] + b1[...]
    t = jnp.maximum(h2_ref[...] @ w[:_D, :] + gv, 0.0)
    z_ref[...] = t @ w2[...] + b2[...] + lm_ref[...]


def _k_softmax(z_ref, o_ref):
    z = z_ref[...]
    m = jnp.max(z)
    e = jnp.exp(z - m)
    o_ref[...] = e / jnp.sum(e)


def _full(shape):
    return pl.BlockSpec(shape, lambda i: tuple(0 for _ in shape))


def _rows(width):
    return pl.BlockSpec((_R, width), lambda i: (i, 0))


_GRID = (pl.cdiv(_N, _R),)


def _tc_init(x, w1, b1, w2, b2, wn):
    return pl.pallas_call(
        _k_init,
        grid=_GRID,
        in_specs=[_rows(_D), _full((_D, _D)), _full((1, _D)),
                  _full((_D, _D)), _full((1, _D)), _full((_D, _D))],
        out_specs=[_rows(_D), _rows(_D)],
        out_shape=[jax.ShapeDtypeStruct((_N, _D), _F32)] * 2,
    )(x, w1, b1, w2, b2, wn)


def _tc_mid0(h, p, cnts, wself, b, w1, b1, w2, b2, wn1):
    return pl.pallas_call(
        _k_mid0,
        grid=_GRID,
        in_specs=[_rows(_D),
                  pl.BlockSpec((_NC, _R, _D), lambda i: (0, i, 0)),
                  pl.BlockSpec((_NC, _NS, _R), lambda i: (0, 0, i)),
                  _full((_D, _D)), _full((1, _D)), _full((_D, _D)),
                  _full((1, _D)), _full((_D, _D)), _full((1, _D)),
                  _full((_D, _D))],
        out_specs=[_rows(_D), _rows(_D), _rows(_D)],
        out_shape=[jax.ShapeDtypeStruct((_N, _D), _F32)] * 3,
    )(h, p, cnts, wself, b, w1, b1, w2, b2, wn1)


def _tc_mid1(h1, q, inv, wself, b, w1, b1, w2, b2):
    return pl.pallas_call(
        _k_mid1,
        grid=_GRID,
        in_specs=[_rows(_D),
                  pl.BlockSpec((_NC, _R, _D), lambda i: (0, i, 0)),
                  _rows(_D),
                  _full((_D, _D)), _full((1, _D)), _full((_D, _D)),
                  _full((1, _D)), _full((_D, _D)), _full((1, _D))],
        out_specs=[_rows(_D), pl.BlockSpec((1, _D), lambda i: (0, 0))],
        out_shape=[jax.ShapeDtypeStruct((_N, _D), _F32),
                   jax.ShapeDtypeStruct((1, _D), _F32)],
    )(h1, q, inv, wself, b, w1, b1, w2, b2)


def _tc_final(h2, g, w1, b1, w2, b2, lm):
    return pl.pallas_call(
        _k_final,
        grid=_GRID,
        in_specs=[_rows(_D), _full((1, _D)), _full((2 * _D, 2 * _D)),
                  _full((1, 2 * _D)), _full((2 * _D, 1)), _full((1, 1)),
                  _rows(1)],
        out_specs=_rows(1),
        out_shape=jax.ShapeDtypeStruct((_N, 1), _F32),
    )(h2, g, w1, b1, w2, b2, lm)


_NPAD = 79 * 128


def _tc_softmax(zp):
    return pl.pallas_call(
        _k_softmax,
        in_specs=[pl.BlockSpec((79, 128), lambda: (0, 0))],
        out_specs=pl.BlockSpec((79, 128), lambda: (0, 0)),
        out_shape=jax.ShapeDtypeStruct((79, 128), _F32),
    )(zp)


def kernel(x, edge_index, op_mask, f_init_W1, f_init_b1, f_init_W2, f_init_b2,
           sage0_Wself, sage0_Wneigh, sage0_b, fs0_W1, fs0_b1, fs0_W2, fs0_b2,
           sage1_Wself, sage1_Wneigh, sage1_b, fs1_W1, fs1_b1, fs1_W2, fs1_b2,
           final_W1, final_b1, final_W2, final_b2):
    src = edge_index[0].astype(jnp.int32)
    dst = edge_index[1].astype(jnp.int32)
    row = lambda v: v.reshape(1, -1)
    zrows = jnp.zeros((_RPT, _D), _F32)
    zcnt = jnp.zeros((_NACC,), _F32)

    h, t0 = _tc_init(x, f_init_W1, row(f_init_b1), f_init_W2, row(f_init_b2),
                     sage0_Wneigh)

    p0, cnts = _sc_scatter_cnt(t0, src, dst, zrows, zcnt)

    h1, t1, inv = _tc_mid0(h, p0, cnts, sage0_Wself, row(sage0_b), fs0_W1,
                           row(fs0_b1), fs0_W2, row(fs0_b2), sage1_Wneigh)

    (q,) = _sc_scatter(t1, src, dst, zrows, zcnt)

    h2, cs = _tc_mid1(h1, q, inv, sage1_Wself, row(sage1_b), fs1_W1,
                      row(fs1_b1), fs1_W2, row(fs1_b2))

    g = cs / np.float32(_N)
    lm = jnp.log(op_mask.astype(_F32)).reshape(-1, 1)
    z = _tc_final(h2, g, final_W1, row(final_b1), final_W2,
                  final_b2.reshape(1, 1), lm)

    zp = jnp.pad(z.reshape(-1), (0, _NPAD - _N),
                 constant_values=-1e30).reshape(79, 128)
    p = _tc_softmax(zp)
    return p.reshape(-1)[:_N].reshape(_N, 1)
